# Initial kernel scaffold; baseline (speedup 1.0000x reference)
import jax
import jax.numpy as jnp
from jax.experimental import pallas as pl


def kernel(node_ids, node_features):
    raise NotImplementedError("write your pallas kernel here")


# speedup vs baseline: 4.5360x; 4.5360x over previous
import functools

import jax
import jax.numpy as jnp
from jax import lax
from jax.experimental import pallas as pl
from jax.experimental.pallas import tpu as pltpu
from jax.experimental.pallas import tpu_sc as plsc

NC = 2
NS = 16
NW = NC * NS

B, L = 4096, 50
D = 128
TOTAL = B * L
PER_W = TOTAL // NW
CHUNK = 128
NCHUNK = PER_W // CHUNK

_mesh = plsc.VectorSubcoreMesh(
    core_axis_name="c", subcore_axis_name="s", num_cores=NC, num_subcores=NS
)


@functools.partial(
    pl.kernel,
    out_type=jax.ShapeDtypeStruct((TOTAL, D), jnp.float32),
    mesh=_mesh,
    scratch_types=[
        pltpu.VMEM((NCHUNK, CHUNK), jnp.int32),
        pltpu.VMEM((CHUNK, D), jnp.float32),
        pltpu.SemaphoreType.DMA,
    ],
)
def _gather_kernel(ids_hbm, table_hbm, out_hbm, idx_v, rows_v, sem):
    wid = lax.axis_index("s") * NC + lax.axis_index("c")
    base = wid * PER_W
    pltpu.sync_copy(ids_hbm.at[wid], idx_v)

    def chunk_body(j, carry):
        pltpu.async_copy(table_hbm.at[idx_v.at[j]], rows_v, sem).wait()
        pltpu.sync_copy(rows_v, out_hbm.at[pl.ds(base + j * CHUNK, CHUNK)])
        return carry

    lax.fori_loop(0, NCHUNK, chunk_body, 0)


def kernel(node_ids, node_features):
    ids = node_ids.reshape(NW, NCHUNK, CHUNK)
    out = _gather_kernel(ids, node_features)
    return out.reshape(B, L * D)


# speedup vs baseline: 5.2874x; 1.1656x over previous
import functools

import jax
import jax.numpy as jnp
from jax import lax
from jax.experimental import pallas as pl
from jax.experimental.pallas import tpu as pltpu
from jax.experimental.pallas import tpu_sc as plsc

NC = 2
NS = 16
NW = NC * NS

B, L = 4096, 50
D = 128
TOTAL = B * L
PER_W = TOTAL // NW
CHUNK = 80
NCHUNK = PER_W // CHUNK
K = 5
NGROUP = NCHUNK // K
NPAIR = NGROUP // 2 - 1

_mesh = plsc.VectorSubcoreMesh(
    core_axis_name="c", subcore_axis_name="s", num_cores=NC, num_subcores=NS
)


@functools.partial(
    pl.kernel,
    out_type=jax.ShapeDtypeStruct((TOTAL, D), jnp.float32),
    mesh=_mesh,
    scratch_types=[
        pltpu.VMEM((NCHUNK, CHUNK), jnp.int32),
        pltpu.VMEM((2 * K, CHUNK, D), jnp.float32),
        pltpu.SemaphoreType.DMA,
        pltpu.SemaphoreType.DMA,
        pltpu.SemaphoreType.DMA,
        pltpu.SemaphoreType.DMA,
    ],
)
def _gather_kernel(ids_hbm, table_hbm, out_hbm, idx_v, bufs, gsa, gsb, wsa, wsb):
    wid = lax.axis_index("s") * NC + lax.axis_index("c")
    base = wid * PER_W
    pltpu.sync_copy(ids_hbm.at[wid], idx_v)

    gsem = (gsa, gsb)
    wsem = (wsa, wsb)

    def fire_gathers(g, s):
        for b in range(K):
            pltpu.async_copy(
                table_hbm.at[idx_v.at[g * K + b]], bufs.at[s * K + b], gsem[s]
            )

    def drain_gathers(g, s):
        for b in range(K):
            pltpu.make_async_copy(
                table_hbm.at[idx_v.at[g * K + b]], bufs.at[s * K + b], gsem[s]
            ).wait()

    def fire_wbs(g, s):
        for b in range(K):
            j = g * K + b
            pltpu.async_copy(
                bufs.at[s * K + b],
                out_hbm.at[pl.ds(base + j * CHUNK, CHUNK)],
                wsem[s],
            )

    def drain_wbs(g, s):
        for b in range(K):
            j = g * K + b
            pltpu.make_async_copy(
                bufs.at[s * K + b],
                out_hbm.at[pl.ds(base + j * CHUNK, CHUNK)],
                wsem[s],
            ).wait()

    fire_gathers(0, 0)
    drain_gathers(0, 0)
    fire_wbs(0, 0)
    fire_gathers(1, 1)

    def half(g, s):
        drain_gathers(g, s)
        fire_wbs(g, s)
        drain_wbs(g - 1, 1 - s)
        fire_gathers(g + 1, 1 - s)

    def pair(gg, carry):
        half(2 * gg + 1, 1)
        half(2 * gg + 2, 0)
        return carry

    lax.fori_loop(0, NPAIR, pair, 0)

    g_last = NGROUP - 1
    drain_gathers(g_last, 1)
    fire_wbs(g_last, 1)
    drain_wbs(g_last - 1, 0)
    drain_wbs(g_last, 1)


def kernel(node_ids, node_features):
    ids = node_ids.reshape(NW, NCHUNK, CHUNK)
    out = _gather_kernel(ids, node_features)
    return out.reshape(B, L * D)


# speedup vs baseline: 10.5476x; 1.9949x over previous
import functools

import jax
import jax.numpy as jnp
from jax import lax
from jax.experimental import pallas as pl
from jax.experimental.pallas import tpu as pltpu
from jax.experimental.pallas import tpu_sc as plsc

NC = 2
NS = 16
NW = NC * NS

B, L = 4096, 50
D = 128
TOTAL = B * L
PER_W = TOTAL // NW
CHUNK = 64
NCHUNK = PER_W // CHUNK
K = 5
NGROUP = NCHUNK // K
NPAIR = NGROUP // 2 - 1

_mesh = plsc.VectorSubcoreMesh(
    core_axis_name="c", subcore_axis_name="s", num_cores=NC, num_subcores=NS
)


@functools.partial(
    pl.kernel,
    out_type=jax.ShapeDtypeStruct((B, L * D), jnp.float32),
    mesh=_mesh,
    compiler_params=pltpu.CompilerParams(use_tc_tiling_on_sc=True),
    scratch_types=[
        pltpu.VMEM((NCHUNK, CHUNK), jnp.int32),
        pltpu.VMEM((2 * K, CHUNK, D), jnp.float32),
        pltpu.SemaphoreType.DMA,
        pltpu.SemaphoreType.DMA,
        pltpu.SemaphoreType.DMA,
        pltpu.SemaphoreType.DMA,
    ],
)
def _gather_kernel(ids_hbm, table_hbm, out_hbm, idx_v, bufs, gsa, gsb, wsa, wsb):
    wid = lax.axis_index("s") * NC + lax.axis_index("c")
    row_base = wid * (B // NW)
    pltpu.sync_copy(ids_hbm.at[wid], idx_v)

    gsem = (gsa, gsb)
    wsem = (wsa, wsb)

    def out_window(j):
        r0 = row_base + CHUNK * lax.rem(j, 2)
        c0 = D * lax.div(j, 2)
        return out_hbm.at[pl.ds(r0, CHUNK), pl.ds(c0, D)]

    def fire_gathers(g, s):
        for b in range(K):
            pltpu.async_copy(
                table_hbm.at[idx_v.at[g * K + b]], bufs.at[s * K + b], gsem[s]
            )

    def drain_gathers(g, s):
        for b in range(K):
            pltpu.make_async_copy(
                table_hbm.at[idx_v.at[g * K + b]], bufs.at[s * K + b], gsem[s]
            ).wait()

    def fire_wbs(g, s):
        for b in range(K):
            pltpu.async_copy(bufs.at[s * K + b], out_window(g * K + b), wsem[s])

    def drain_wbs(g, s):
        for b in range(K):
            pltpu.make_async_copy(
                bufs.at[s * K + b], out_window(g * K + b), wsem[s]
            ).wait()

    fire_gathers(0, 0)
    drain_gathers(0, 0)
    fire_wbs(0, 0)
    fire_gathers(1, 1)

    def half(g, s):
        drain_gathers(g, s)
        fire_wbs(g, s)
        drain_wbs(g - 1, 1 - s)
        fire_gathers(g + 1, 1 - s)

    def pair(gg, carry):
        half(2 * gg + 1, 1)
        half(2 * gg + 2, 0)
        return carry

    lax.fori_loop(0, NPAIR, pair, 0)

    g_last = NGROUP - 1
    drain_gathers(g_last, 1)
    fire_wbs(g_last, 1)
    drain_wbs(g_last - 1, 0)
    drain_wbs(g_last, 1)


def kernel(node_ids, node_features):
    ids = node_ids.reshape(NW, 2, CHUNK, L)
    ids = ids.transpose(0, 3, 1, 2)
    ids = ids.reshape(NW, NCHUNK, CHUNK)
    return _gather_kernel(ids, node_features)


# speedup vs baseline: 11.2262x; 1.0643x over previous
import functools

import jax
import jax.numpy as jnp
from jax import lax
from jax.experimental import pallas as pl
from jax.experimental.pallas import tpu as pltpu
from jax.experimental.pallas import tpu_sc as plsc

NC = 2
NS = 16
NW = NC * NS

B, L = 4096, 50
D = 128
TOTAL = B * L
PER_W = TOTAL // NW
CHUNK = 128
NCHUNK = PER_W // CHUNK
NBUF = 5
GAHEAD = 3

_mesh = plsc.VectorSubcoreMesh(
    core_axis_name="c", subcore_axis_name="s", num_cores=NC, num_subcores=NS
)


@functools.partial(
    pl.kernel,
    out_type=jax.ShapeDtypeStruct((B, L * D), jnp.float32),
    mesh=_mesh,
    compiler_params=pltpu.CompilerParams(use_tc_tiling_on_sc=True),
    scratch_types=[
        pltpu.VMEM((NCHUNK, CHUNK), jnp.int32),
        pltpu.VMEM((NBUF, CHUNK, D), jnp.float32),
        tuple(pltpu.SemaphoreType.DMA for _ in range(NBUF)),
        tuple(pltpu.SemaphoreType.DMA for _ in range(NBUF)),
    ],
)
def _gather_kernel(ids_hbm, table_hbm, out_hbm, idx_v, bufs, gsem, wsem):
    wid = lax.axis_index("s") * NC + lax.axis_index("c")
    row_base = wid * (B // NW)
    pltpu.sync_copy(ids_hbm.at[wid], idx_v)

    def out_window(j):
        return out_hbm.at[pl.ds(row_base, CHUNK), pl.ds(D * j, D)]

    def fire_gather(j, s):
        pltpu.async_copy(table_hbm.at[idx_v.at[j]], bufs.at[s], gsem[s])

    def drain_gather(j, s):
        pltpu.make_async_copy(table_hbm.at[idx_v.at[j]], bufs.at[s], gsem[s]).wait()

    def fire_wb(j, s):
        pltpu.async_copy(bufs.at[s], out_window(j), wsem[s])

    def drain_wb(j, s):
        pltpu.make_async_copy(bufs.at[s], out_window(j), wsem[s]).wait()

    def step(j, s, drain_prev=True, fire_next=True):
        drain_gather(j, s)
        fire_wb(j, s)
        if drain_prev:
            drain_wb(j - (NBUF - GAHEAD), (s + GAHEAD) % NBUF)
        if fire_next:

            @pl.when(j + GAHEAD < NCHUNK)
            def _():
                fire_gather(j + GAHEAD, (s + GAHEAD) % NBUF)

    for s in range(GAHEAD):
        fire_gather(s, s)
    step(0, 0, drain_prev=False)
    step(1, 1, drain_prev=False)
    for j in range(2, NBUF):
        step(j, j)

    def block(bb, carry):
        j0 = NBUF * bb + NBUF
        for s in range(NBUF):
            step(j0 + s, s)
        return carry

    lax.fori_loop(0, NCHUNK // NBUF - 1, block, 0)

    drain_wb(NCHUNK - 2, (NCHUNK - 2) % NBUF)
    drain_wb(NCHUNK - 1, (NCHUNK - 1) % NBUF)


def kernel(node_ids, node_features):
    ids = node_ids.reshape(NW, CHUNK, L).transpose(0, 2, 1)
    return _gather_kernel(ids, node_features)


# speedup vs baseline: 11.2844x; 1.0052x over previous
import functools

import jax
import jax.numpy as jnp
from jax import lax
from jax.experimental import pallas as pl
from jax.experimental.pallas import tpu as pltpu
from jax.experimental.pallas import tpu_sc as plsc

NC = 2
NS = 16
NW = NC * NS

B, L = 4096, 50
D = 128
TOTAL = B * L
PER_W = TOTAL // NW
CHUNK = 128
NCHUNK = PER_W // CHUNK
NBUF = 6
GAHEAD = 4

_mesh = plsc.VectorSubcoreMesh(
    core_axis_name="c", subcore_axis_name="s", num_cores=NC, num_subcores=NS
)


@functools.partial(
    pl.kernel,
    out_type=jax.ShapeDtypeStruct((B, L * D), jnp.float32),
    mesh=_mesh,
    compiler_params=pltpu.CompilerParams(use_tc_tiling_on_sc=True),
    scratch_types=[
        pltpu.VMEM((NCHUNK, CHUNK), jnp.int32),
        pltpu.VMEM((NBUF, CHUNK, D), jnp.float32),
        tuple(pltpu.SemaphoreType.DMA for _ in range(NBUF)),
        tuple(pltpu.SemaphoreType.DMA for _ in range(NBUF)),
    ],
)
def _gather_kernel(ids_hbm, table_hbm, out_hbm, idx_v, bufs, gsem, wsem):
    wid = lax.axis_index("s") * NC + lax.axis_index("c")
    row_base = wid * (B // NW)
    pltpu.sync_copy(ids_hbm.at[wid], idx_v)

    def out_window(j):
        return out_hbm.at[pl.ds(row_base, CHUNK), pl.ds(D * j, D)]

    def fire_gather(j, s):
        pltpu.async_copy(table_hbm.at[idx_v.at[j]], bufs.at[s], gsem[s])

    def drain_gather(j, s):
        pltpu.make_async_copy(table_hbm.at[idx_v.at[j]], bufs.at[s], gsem[s]).wait()

    def fire_wb(j, s):
        pltpu.async_copy(bufs.at[s], out_window(j), wsem[s])

    def drain_wb(j, s):
        pltpu.make_async_copy(bufs.at[s], out_window(j), wsem[s]).wait()

    def step(j, s, drain_prev=True, fire_next=True):
        drain_gather(j, s)
        fire_wb(j, s)
        if drain_prev:
            drain_wb(j - (NBUF - GAHEAD), (s + GAHEAD) % NBUF)
        if fire_next:

            @pl.when(j + GAHEAD < NCHUNK)
            def _():
                fire_gather(j + GAHEAD, (s + GAHEAD) % NBUF)

    for s in range(GAHEAD):
        fire_gather(s, s)
    step(0, 0, drain_prev=False)
    step(1, 1, drain_prev=False)

    def block(bb, carry):
        j0 = NBUF * bb + 2
        for k in range(NBUF):
            step(j0 + k, (2 + k) % NBUF)
        return carry

    lax.fori_loop(0, (NCHUNK - 2) // NBUF, block, 0)

    drain_wb(NCHUNK - 2, (NCHUNK - 2) % NBUF)
    drain_wb(NCHUNK - 1, (NCHUNK - 1) % NBUF)


def kernel(node_ids, node_features):
    ids = node_ids.reshape(NW, CHUNK, L).transpose(0, 2, 1)
    return _gather_kernel(ids, node_features)


# speedup vs baseline: 11.3627x; 1.0069x over previous
import functools

import jax
import jax.numpy as jnp
from jax import lax
from jax.experimental import pallas as pl
from jax.experimental.pallas import tpu as pltpu
from jax.experimental.pallas import tpu_sc as plsc

NC = 2
NS = 16
NW = NC * NS

B, L = 4096, 50
D = 128
TOTAL = B * L
PER_W = TOTAL // NW
CHUNK = 128
NCHUNK = PER_W // CHUNK
NBUF = 6
GAHEAD = 4

_mesh = plsc.VectorSubcoreMesh(
    core_axis_name="c", subcore_axis_name="s", num_cores=NC, num_subcores=NS
)


@functools.partial(
    pl.kernel,
    out_type=jax.ShapeDtypeStruct((B, L * D), jnp.float32),
    mesh=_mesh,
    compiler_params=pltpu.CompilerParams(use_tc_tiling_on_sc=True),
    scratch_types=[
        pltpu.VMEM((NCHUNK, CHUNK), jnp.int32),
        pltpu.VMEM((NBUF, CHUNK, D), jnp.float32),
        tuple(pltpu.SemaphoreType.DMA for _ in range(NBUF)),
        tuple(pltpu.SemaphoreType.DMA for _ in range(NBUF)),
    ],
)
def _gather_kernel(ids_hbm, table_hbm, out_hbm, idx_v, bufs, gsem, wsem):
    wid = lax.axis_index("s") * NC + lax.axis_index("c")
    row_base = wid * (B // NW)
    pltpu.sync_copy(ids_hbm.at[pl.ds(0, GAHEAD), wid], idx_v.at[pl.ds(0, GAHEAD)])

    def out_window(j):
        return out_hbm.at[pl.ds(row_base, CHUNK), pl.ds(D * j, D)]

    def fire_gather(j, s):
        pltpu.async_copy(table_hbm.at[idx_v.at[j]], bufs.at[s], gsem[s])

    def drain_gather(j, s):
        pltpu.make_async_copy(table_hbm.at[idx_v.at[j]], bufs.at[s], gsem[s]).wait()

    def fire_wb(j, s):
        pltpu.async_copy(bufs.at[s], out_window(j), wsem[s])

    def drain_wb(j, s):
        pltpu.make_async_copy(bufs.at[s], out_window(j), wsem[s]).wait()

    def step(j, s, drain_prev=True, fire_next=True):
        drain_gather(j, s)
        fire_wb(j, s)
        if drain_prev:
            drain_wb(j - (NBUF - GAHEAD), (s + GAHEAD) % NBUF)
        if fire_next:

            @pl.when(j + GAHEAD < NCHUNK)
            def _():
                fire_gather(j + GAHEAD, (s + GAHEAD) % NBUF)

    for s in range(GAHEAD):
        fire_gather(s, s)
    pltpu.sync_copy(
        ids_hbm.at[pl.ds(GAHEAD, NCHUNK - GAHEAD), wid],
        idx_v.at[pl.ds(GAHEAD, NCHUNK - GAHEAD)],
    )
    step(0, 0, drain_prev=False)
    step(1, 1, drain_prev=False)

    def block(bb, carry):
        j0 = NBUF * bb + 2
        for k in range(NBUF):
            step(j0 + k, (2 + k) % NBUF)
        return carry

    lax.fori_loop(0, (NCHUNK - 2) // NBUF, block, 0)

    drain_wb(NCHUNK - 2, (NCHUNK - 2) % NBUF)
    drain_wb(NCHUNK - 1, (NCHUNK - 1) % NBUF)


def kernel(node_ids, node_features):
    ids = node_ids.T.reshape(L, NW, CHUNK)
    return _gather_kernel(ids, node_features)
